# Initial kernel scaffold; baseline (speedup 1.0000x reference)
import functools

import jax
import jax.numpy as jnp
from jax import lax
from jax.experimental import pallas as pl
from jax.experimental.pallas import tpu as pltpu
from jax.experimental.pallas import tpu_sc as plsc

N = 10000
E = 320000
H = 64

NC = 2
NS = 16
NW = NC * NS
EPW = E // NW
CHUNK = 80
NCHUNK = EPW // CHUNK
ROWS_PER_TILE = N // NS
ZROWS = 125


def _sc_edge_aggregate(xproj, ceproj, senders, receivers):
  mesh = plsc.VectorSubcoreMesh(core_axis_name="c", subcore_axis_name="s")

  @functools.partial(
      pl.kernel,
      out_type=jax.ShapeDtypeStruct((NC, N, H), jnp.float32),
      mesh=mesh,
      scratch_types=[
          pltpu.VMEM((1, CHUNK), jnp.int32),
          pltpu.VMEM((1, CHUNK), jnp.int32),
          pltpu.VMEM((CHUNK, H), jnp.float32),
          pltpu.VMEM((CHUNK, H), jnp.float32),
          pltpu.VMEM((ZROWS, H), jnp.float32),
          pltpu.VMEM_SHARED((N, H), jnp.float32),
          pltpu.SemaphoreType.DMA,
      ],
  )
  def k(xproj_hbm, ceproj_hbm, snd_hbm, rcv_hbm, out_hbm,
        sidx, ridx, ce, xs, zbuf, acc, sem):
    c = lax.axis_index("c")
    s = lax.axis_index("s")
    wid = s * NC + c

    def zrow(r, carry):
      for q in range(H // 16):
        zbuf[r, pl.ds(q * 16, 16)] = jnp.zeros((16,), jnp.float32)
      return carry
    lax.fori_loop(0, ZROWS, zrow, 0)
    for kk in range(ROWS_PER_TILE // ZROWS):
      pltpu.sync_copy(zbuf, acc.at[pl.ds(s * ROWS_PER_TILE + kk * ZROWS, ZROWS)])
    plsc.subcore_barrier()

    def step(j, carry):
      base = wid * EPW + j * CHUNK
      pltpu.sync_copy(snd_hbm.at[pl.ds(base, CHUNK)], sidx.at[0])
      pltpu.sync_copy(rcv_hbm.at[pl.ds(base, CHUNK)], ridx.at[0])
      pltpu.async_copy(xproj_hbm.at[sidx.at[0]], xs, sem).wait()
      pltpu.sync_copy(ceproj_hbm.at[pl.ds(base, CHUNK)], ce)

      def row(r, rc):
        for q in range(H // 16):
          sl = pl.ds(q * 16, 16)
          ce[r, sl] = jnp.maximum(ce[r, sl] + xs[r, sl], 0.0)
        return rc
      lax.fori_loop(0, CHUNK, row, 0)

      pltpu.sync_copy(ce, acc.at[ridx.at[0]], add=True)
      return carry
    lax.fori_loop(0, NCHUNK, step, 0)
    plsc.subcore_barrier()

    pltpu.sync_copy(acc.at[pl.ds(s * ROWS_PER_TILE, ROWS_PER_TILE)],
                    out_hbm.at[c, pl.ds(s * ROWS_PER_TILE, ROWS_PER_TILE)])

  return k(xproj, ceproj, senders, receivers)


def _xproj_body(x_ref, wx_ref, g_ref, wg_ref, b_ref, o_ref):
  const = jnp.dot(g_ref[...], wg_ref[...], preferred_element_type=jnp.float32)
  o_ref[...] = (jnp.dot(x_ref[...], wx_ref[...],
                        preferred_element_type=jnp.float32)
                + const + b_ref[...])


def _ceproj_body(e_ref, we_ref, o_ref):
  o_ref[...] = jnp.dot(e_ref[...], we_ref[...],
                       preferred_element_type=jnp.float32)


def _node1_body(aggp_ref, x_ref, wa_ref, wx_ref, g_ref, wg_ref, b_ref,
                agg_ref, hv_ref, sums_ref, acc_ref):
  i = pl.program_id(0)
  agg = aggp_ref[0] + aggp_ref[1]
  agg_ref[...] = agg
  const = jnp.dot(g_ref[...], wg_ref[...], preferred_element_type=jnp.float32)
  hv = jnp.maximum(
      jnp.dot(agg, wa_ref[...], preferred_element_type=jnp.float32)
      + jnp.dot(x_ref[...], wx_ref[...], preferred_element_type=jnp.float32)
      + const + b_ref[...], 0.0)
  hv_ref[...] = hv
  part = jnp.concatenate(
      [jnp.sum(agg, axis=0, keepdims=True),
       jnp.sum(hv, axis=0, keepdims=True)], axis=0)

  @pl.when(i == 0)
  def _():
    acc_ref[...] = jnp.zeros_like(acc_ref)

  acc_ref[0:2, 0:H] += part

  @pl.when(i == pl.num_programs(0) - 1)
  def _():
    sums_ref[...] = acc_ref[0:2, 0:H]


def _node2_body(agg_ref, hv_ref, sums_ref, g_ref,
                wg1_ref, wg2_ref, wg3_ref, bgb_ref,
                wa_ref, wv_ref, wgn_ref, b2a_ref, w2b_ref, b2b_ref,
                out_ref, gnew_ref):
  i = pl.program_id(0)
  mean_he = sums_ref[0:1, :] * (1.0 / E)
  mean_hv = sums_ref[1:2, :] * (1.0 / N)
  g_new = jnp.maximum(
      jnp.dot(mean_he, wg1_ref[...], preferred_element_type=jnp.float32)
      + jnp.dot(mean_hv, wg2_ref[...], preferred_element_type=jnp.float32)
      + jnp.dot(g_ref[...], wg3_ref[...], preferred_element_type=jnp.float32)
      + bgb_ref[...], 0.0)
  h2 = jnp.maximum(
      jnp.dot(agg_ref[...], wa_ref[...], preferred_element_type=jnp.float32)
      + jnp.dot(hv_ref[...], wv_ref[...], preferred_element_type=jnp.float32)
      + jnp.dot(g_new, wgn_ref[...], preferred_element_type=jnp.float32)
      + b2a_ref[...], 0.0)
  out_ref[...] = (jnp.dot(h2, w2b_ref[...], preferred_element_type=jnp.float32)
                  + b2b_ref[...])

  @pl.when(i == 0)
  def _():
    gnew_ref[...] = g_new


def _full(shape):
  nd = len(shape)
  return pl.BlockSpec(shape, lambda i: (0,) * nd)


def kernel(cat_x, cat_e, edge_index, global_attr, W_eb, b_eb, W_nb, b_nb,
           W_gb, b_gb, W_n2a, b_n2a, W_n2b, b_n2b):
  IN_X = cat_x.shape[1]
  IN_E = cat_e.shape[1]
  G = global_attr.shape[0]
  senders = edge_index[0]
  receivers = edge_index[1]
  g_row = global_attr.reshape(1, G)

  W_eb_e = W_eb[:IN_E]
  W_eb_x = W_eb[IN_E:IN_E + IN_X]
  W_eb_g = W_eb[IN_E + IN_X:]

  BN = 2000
  xproj = pl.pallas_call(
      _xproj_body,
      grid=(N // BN,),
      in_specs=[pl.BlockSpec((BN, IN_X), lambda i: (i, 0)),
                _full((IN_X, H)), _full((1, G)), _full((G, H)), _full((1, H))],
      out_specs=pl.BlockSpec((BN, H), lambda i: (i, 0)),
      out_shape=jax.ShapeDtypeStruct((N, H), jnp.float32),
  )(cat_x, W_eb_x, g_row, W_eb_g, b_eb.reshape(1, H))

  BE = 8000
  ceproj = pl.pallas_call(
      _ceproj_body,
      grid=(E // BE,),
      in_specs=[pl.BlockSpec((BE, IN_E), lambda i: (i, 0)),
                _full((IN_E, H))],
      out_specs=pl.BlockSpec((BE, H), lambda i: (i, 0)),
      out_shape=jax.ShapeDtypeStruct((E, H), jnp.float32),
  )(cat_e, W_eb_e)

  agg_parts = _sc_edge_aggregate(xproj, ceproj, senders, receivers)

  W_nb_a = W_nb[:H]
  W_nb_x = W_nb[H:H + IN_X]
  W_nb_g = W_nb[H + IN_X:]
  agg, h_v, sums = pl.pallas_call(
      _node1_body,
      grid=(N // BN,),
      in_specs=[pl.BlockSpec((NC, BN, H), lambda i: (0, i, 0)),
                pl.BlockSpec((BN, IN_X), lambda i: (i, 0)),
                _full((H, H)), _full((IN_X, H)), _full((1, G)), _full((G, H)),
                _full((1, H))],
      out_specs=[pl.BlockSpec((BN, H), lambda i: (i, 0)),
                 pl.BlockSpec((BN, H), lambda i: (i, 0)),
                 _full((2, H))],
      out_shape=[jax.ShapeDtypeStruct((N, H), jnp.float32),
                 jax.ShapeDtypeStruct((N, H), jnp.float32),
                 jax.ShapeDtypeStruct((2, H), jnp.float32)],
      scratch_shapes=[pltpu.VMEM((8, 128), jnp.float32)],
  )(agg_parts, cat_x, W_nb_a, W_nb_x, g_row, W_nb_g, b_nb.reshape(1, H))

  OUT = W_n2b.shape[1]
  W_gb1 = W_gb[:H]
  W_gb2 = W_gb[H:2 * H]
  W_gb3 = W_gb[2 * H:]
  W_n2a_a = W_n2a[:H]
  W_n2a_v = W_n2a[H:2 * H]
  W_n2a_g = W_n2a[2 * H:]
  out_nodes, g_new = pl.pallas_call(
      _node2_body,
      grid=(N // BN,),
      in_specs=[pl.BlockSpec((BN, H), lambda i: (i, 0)),
                pl.BlockSpec((BN, H), lambda i: (i, 0)),
                _full((2, H)), _full((1, G)),
                _full((H, G)), _full((H, G)), _full((G, G)), _full((1, G)),
                _full((H, H)), _full((H, H)), _full((G, H)), _full((1, H)),
                _full((H, OUT)), _full((1, OUT))],
      out_specs=[pl.BlockSpec((BN, OUT), lambda i: (i, 0)),
                 _full((1, G))],
      out_shape=[jax.ShapeDtypeStruct((N, OUT), jnp.float32),
                 jax.ShapeDtypeStruct((1, G), jnp.float32)],
  )(agg, h_v, sums, g_row, W_gb1, W_gb2, W_gb3, b_gb.reshape(1, G),
    W_n2a_a, W_n2a_v, W_n2a_g, b_n2a.reshape(1, H),
    W_n2b, b_n2b.reshape(1, OUT))

  return (out_nodes, g_new.reshape(G))


# speedup vs baseline: 2.9661x; 2.9661x over previous
import functools

import jax
import jax.numpy as jnp
from jax import lax
from jax.experimental import pallas as pl
from jax.experimental.pallas import tpu as pltpu
from jax.experimental.pallas import tpu_sc as plsc

N = 10000
E = 320000
H = 64
HP = 128

NC = 2
NS = 16
NW = NC * NS
EPW = E // NW
CHUNK = 80
NCHUNK = EPW // CHUNK
N_PAD = 10240
ROWS_PER_TILE = N_PAD // NS
ZROWS = 128


def _sc_edge_aggregate(xproj, ceproj, senders, receivers):
  mesh = plsc.VectorSubcoreMesh(core_axis_name="c", subcore_axis_name="s")

  @functools.partial(
      pl.kernel,
      out_type=jax.ShapeDtypeStruct((NC, N_PAD, HP), jnp.float32),
      mesh=mesh,
      scratch_types=[
          pltpu.VMEM((1, CHUNK), jnp.int32),
          pltpu.VMEM((1, CHUNK), jnp.int32),
          pltpu.VMEM((CHUNK, H), jnp.float32),
          pltpu.VMEM((CHUNK, HP), jnp.float32),
          pltpu.VMEM((CHUNK, HP), jnp.float32),
          pltpu.VMEM((ZROWS, HP), jnp.float32),
          pltpu.VMEM_SHARED((N_PAD, HP), jnp.float32),
          pltpu.SemaphoreType.DMA,
      ],
  )
  def k(xproj_hbm, ceproj_hbm, snd_hbm, rcv_hbm, out_hbm,
        sidx, ridx, ce, xs, he, zbuf, acc, sem):
    c = lax.axis_index("c")
    s = lax.axis_index("s")
    wid = s * NC + c

    def zrow(r, carry):
      for q in range(HP // 16):
        zbuf[r, pl.ds(q * 16, 16)] = jnp.zeros((16,), jnp.float32)
      return carry
    lax.fori_loop(0, ZROWS, zrow, 0)

    def zhe(r, carry):
      for q in range(HP // 16):
        he[r, pl.ds(q * 16, 16)] = jnp.zeros((16,), jnp.float32)
      return carry
    lax.fori_loop(0, CHUNK, zhe, 0)

    for kk in range(ROWS_PER_TILE // ZROWS):
      pltpu.sync_copy(zbuf, acc.at[pl.ds(s * ROWS_PER_TILE + kk * ZROWS, ZROWS)])
    plsc.subcore_barrier()

    def step(j, carry):
      base = wid * EPW + j * CHUNK
      pltpu.sync_copy(snd_hbm.at[pl.ds(base, CHUNK)], sidx.at[0])
      pltpu.sync_copy(rcv_hbm.at[pl.ds(base, CHUNK)], ridx.at[0])
      pltpu.async_copy(xproj_hbm.at[sidx.at[0]], xs, sem).wait()
      pltpu.sync_copy(ceproj_hbm.at[pl.ds(base, CHUNK)], ce)

      def row(r, rc):
        for q in range(H // 16):
          sl = pl.ds(q * 16, 16)
          he[r, sl] = jnp.maximum(ce[r, sl] + xs[r, sl], 0.0)
        return rc
      lax.fori_loop(0, CHUNK, row, 0)

      pltpu.sync_copy(he, acc.at[ridx.at[0]], add=True)
      return carry
    lax.fori_loop(0, NCHUNK, step, 0)
    plsc.subcore_barrier()

    pltpu.sync_copy(acc.at[pl.ds(s * ROWS_PER_TILE, ROWS_PER_TILE)],
                    out_hbm.at[c, pl.ds(s * ROWS_PER_TILE, ROWS_PER_TILE)])

  return k(xproj, ceproj, senders, receivers)


def _xproj_body(x_ref, wx_ref, g_ref, wg_ref, b_ref, o_ref):
  const = jnp.dot(g_ref[...], wg_ref[...], preferred_element_type=jnp.float32)
  proj = (jnp.dot(x_ref[...], wx_ref[...], preferred_element_type=jnp.float32)
          + const + b_ref[...])
  o_ref[...] = jnp.concatenate(
      [proj, jnp.zeros((proj.shape[0], HP - H), jnp.float32)], axis=1)


def _ceproj_body(e_ref, we_ref, o_ref):
  o_ref[...] = jnp.dot(e_ref[...], we_ref[...],
                       preferred_element_type=jnp.float32)


def _node1_body(aggp_ref, x_ref, wa_ref, wx_ref, g_ref, wg_ref, b_ref,
                agg_ref, hv_ref, sums_ref, acc_ref):
  i = pl.program_id(0)
  agg = aggp_ref[0, :, :H] + aggp_ref[1, :, :H]
  agg_ref[...] = agg
  const = jnp.dot(g_ref[...], wg_ref[...], preferred_element_type=jnp.float32)
  hv = jnp.maximum(
      jnp.dot(agg, wa_ref[...], preferred_element_type=jnp.float32)
      + jnp.dot(x_ref[...], wx_ref[...], preferred_element_type=jnp.float32)
      + const + b_ref[...], 0.0)
  hv_ref[...] = hv
  part = jnp.concatenate(
      [jnp.sum(agg, axis=0, keepdims=True),
       jnp.sum(hv, axis=0, keepdims=True)], axis=0)

  @pl.when(i == 0)
  def _():
    acc_ref[...] = jnp.zeros_like(acc_ref)

  acc_ref[0:2, 0:H] += part

  @pl.when(i == pl.num_programs(0) - 1)
  def _():
    sums_ref[...] = acc_ref[0:2, 0:H]


def _node2_body(agg_ref, hv_ref, sums_ref, g_ref,
                wg1_ref, wg2_ref, wg3_ref, bgb_ref,
                wa_ref, wv_ref, wgn_ref, b2a_ref, w2b_ref, b2b_ref,
                out_ref, gnew_ref):
  i = pl.program_id(0)
  mean_he = sums_ref[0:1, :] * (1.0 / E)
  mean_hv = sums_ref[1:2, :] * (1.0 / N)
  g_new = jnp.maximum(
      jnp.dot(mean_he, wg1_ref[...], preferred_element_type=jnp.float32)
      + jnp.dot(mean_hv, wg2_ref[...], preferred_element_type=jnp.float32)
      + jnp.dot(g_ref[...], wg3_ref[...], preferred_element_type=jnp.float32)
      + bgb_ref[...], 0.0)
  h2 = jnp.maximum(
      jnp.dot(agg_ref[...], wa_ref[...], preferred_element_type=jnp.float32)
      + jnp.dot(hv_ref[...], wv_ref[...], preferred_element_type=jnp.float32)
      + jnp.dot(g_new, wgn_ref[...], preferred_element_type=jnp.float32)
      + b2a_ref[...], 0.0)
  out_ref[...] = (jnp.dot(h2, w2b_ref[...], preferred_element_type=jnp.float32)
                  + b2b_ref[...])

  @pl.when(i == 0)
  def _():
    gnew_ref[...] = g_new


def _full(shape):
  nd = len(shape)
  return pl.BlockSpec(shape, lambda i: (0,) * nd)


def kernel(cat_x, cat_e, edge_index, global_attr, W_eb, b_eb, W_nb, b_nb,
           W_gb, b_gb, W_n2a, b_n2a, W_n2b, b_n2b):
  IN_X = cat_x.shape[1]
  IN_E = cat_e.shape[1]
  G = global_attr.shape[0]
  senders = edge_index[0]
  receivers = edge_index[1]
  g_row = global_attr.reshape(1, G)

  W_eb_e = W_eb[:IN_E]
  W_eb_x = W_eb[IN_E:IN_E + IN_X]
  W_eb_g = W_eb[IN_E + IN_X:]

  BN = 2000
  xproj = pl.pallas_call(
      _xproj_body,
      grid=(N // BN,),
      in_specs=[pl.BlockSpec((BN, IN_X), lambda i: (i, 0)),
                _full((IN_X, H)), _full((1, G)), _full((G, H)), _full((1, H))],
      out_specs=pl.BlockSpec((BN, HP), lambda i: (i, 0)),
      out_shape=jax.ShapeDtypeStruct((N, HP), jnp.float32),
  )(cat_x, W_eb_x, g_row, W_eb_g, b_eb.reshape(1, H))

  BE = 8000
  ceproj = pl.pallas_call(
      _ceproj_body,
      grid=(E // BE,),
      in_specs=[pl.BlockSpec((BE, IN_E), lambda i: (i, 0)),
                _full((IN_E, H))],
      out_specs=pl.BlockSpec((BE, H), lambda i: (i, 0)),
      out_shape=jax.ShapeDtypeStruct((E, H), jnp.float32),
  )(cat_e, W_eb_e)

  agg_parts = _sc_edge_aggregate(xproj, ceproj, senders, receivers)

  W_nb_a = W_nb[:H]
  W_nb_x = W_nb[H:H + IN_X]
  W_nb_g = W_nb[H + IN_X:]
  agg, h_v, sums = pl.pallas_call(
      _node1_body,
      grid=(N // BN,),
      in_specs=[pl.BlockSpec((NC, BN, HP), lambda i: (0, i, 0)),
                pl.BlockSpec((BN, IN_X), lambda i: (i, 0)),
                _full((H, H)), _full((IN_X, H)), _full((1, G)), _full((G, H)),
                _full((1, H))],
      out_specs=[pl.BlockSpec((BN, H), lambda i: (i, 0)),
                 pl.BlockSpec((BN, H), lambda i: (i, 0)),
                 _full((2, H))],
      out_shape=[jax.ShapeDtypeStruct((N, H), jnp.float32),
                 jax.ShapeDtypeStruct((N, H), jnp.float32),
                 jax.ShapeDtypeStruct((2, H), jnp.float32)],
      scratch_shapes=[pltpu.VMEM((8, 128), jnp.float32)],
  )(agg_parts, cat_x, W_nb_a, W_nb_x, g_row, W_nb_g, b_nb.reshape(1, H))

  OUT = W_n2b.shape[1]
  W_gb1 = W_gb[:H]
  W_gb2 = W_gb[H:2 * H]
  W_gb3 = W_gb[2 * H:]
  W_n2a_a = W_n2a[:H]
  W_n2a_v = W_n2a[H:2 * H]
  W_n2a_g = W_n2a[2 * H:]
  out_nodes, g_new = pl.pallas_call(
      _node2_body,
      grid=(N // BN,),
      in_specs=[pl.BlockSpec((BN, H), lambda i: (i, 0)),
                pl.BlockSpec((BN, H), lambda i: (i, 0)),
                _full((2, H)), _full((1, G)),
                _full((H, G)), _full((H, G)), _full((G, G)), _full((1, G)),
                _full((H, H)), _full((H, H)), _full((G, H)), _full((1, H)),
                _full((H, OUT)), _full((1, OUT))],
      out_specs=[pl.BlockSpec((BN, OUT), lambda i: (i, 0)),
                 _full((1, G))],
      out_shape=[jax.ShapeDtypeStruct((N, OUT), jnp.float32),
                 jax.ShapeDtypeStruct((1, G), jnp.float32)],
  )(agg, h_v, sums, g_row, W_gb1, W_gb2, W_gb3, b_gb.reshape(1, G),
    W_n2a_a, W_n2a_v, W_n2a_g, b_n2a.reshape(1, H),
    W_n2b, b_n2b.reshape(1, OUT))

  return (out_nodes, g_new.reshape(G))


# speedup vs baseline: 3.0685x; 1.0345x over previous
import functools

import jax
import jax.numpy as jnp
from jax import lax
from jax.experimental import pallas as pl
from jax.experimental.pallas import tpu as pltpu
from jax.experimental.pallas import tpu_sc as plsc

N = 10000
E = 320000
H = 64
HP = 128

NC = 2
NS = 16
NW = NC * NS
EPW = E // NW
CHUNK = 80
NCHUNK = EPW // CHUNK
N2 = N // 2
N2_PAD = 5120
ROWS_PER_TILE = N2_PAD // NS
ZROWS = 64


def _sc_edge_aggregate(xproj2, ceproj2, senders, receivers):
  mesh = plsc.VectorSubcoreMesh(core_axis_name="c", subcore_axis_name="s")

  @functools.partial(
      pl.kernel,
      out_type=jax.ShapeDtypeStruct((NC, N2_PAD, HP), jnp.float32),
      mesh=mesh,
      scratch_types=[
          pltpu.VMEM((1, CHUNK), jnp.int32),
          pltpu.VMEM((1, CHUNK), jnp.int32),
          pltpu.VMEM((1, CHUNK), jnp.int32),
          pltpu.VMEM((1, CHUNK), jnp.int32),
          pltpu.VMEM((1, CHUNK), jnp.int32),
          pltpu.VMEM((1, CHUNK), jnp.int32),
          pltpu.VMEM((1, CHUNK), jnp.int32),
          pltpu.VMEM((1, CHUNK), jnp.int32),
          pltpu.VMEM((CHUNK // 2, HP), jnp.float32),
          pltpu.VMEM((CHUNK // 2, HP), jnp.float32),
          pltpu.VMEM((CHUNK, HP), jnp.float32),
          pltpu.VMEM((CHUNK, HP), jnp.float32),
          pltpu.VMEM((CHUNK, HP), jnp.float32),
          pltpu.VMEM((ZROWS, HP), jnp.float32),
          pltpu.VMEM_SHARED((N2_PAD, HP), jnp.float32),
          pltpu.SemaphoreType.DMA,
          pltpu.SemaphoreType.DMA,
          pltpu.SemaphoreType.DMA,
          pltpu.SemaphoreType.DMA,
      ],
  )
  def k(xproj_hbm, ceproj_hbm, snd_hbm, rcv_hbm, out_hbm,
        sidx0, sidx1, ridx0, ridx1, sh0, sh1, rh0, rh1,
        ce0, ce1, xs0, xs1, he, zbuf, acc,
        sem_ce0, sem_ce1, sem_g0, sem_g1):
    c = lax.axis_index("c")
    s = lax.axis_index("s")
    wid = s * NC + c
    ebase = wid * EPW
    pbase = wid * (EPW // 2)
    sidx = (sidx0, sidx1)
    ridx = (ridx0, ridx1)
    sh = (sh0, sh1)
    rh = (rh0, rh1)
    ce = (ce0, ce1)
    xs = (xs0, xs1)
    sem_ce = (sem_ce0, sem_ce1)
    sem_g = (sem_g0, sem_g1)

    def zrow(r, carry):
      for q in range(HP // 16):
        zbuf[r, pl.ds(q * 16, 16)] = jnp.zeros((16,), jnp.float32)
      return carry
    lax.fori_loop(0, ZROWS, zrow, 0)
    for kk in range(ROWS_PER_TILE // ZROWS):
      pltpu.sync_copy(zbuf, acc.at[pl.ds(s * ROWS_PER_TILE + kk * ZROWS, ZROWS)])
    plsc.subcore_barrier()

    def issue(j, p):
      pltpu.sync_copy(snd_hbm.at[pl.ds(ebase + j * CHUNK, CHUNK)], sidx[p].at[0])
      pltpu.sync_copy(rcv_hbm.at[pl.ds(ebase + j * CHUNK, CHUNK)], ridx[p].at[0])
      for q in range(CHUNK // 16):
        sl = pl.ds(q * 16, 16)
        sh[p][0, sl] = lax.shift_right_logical(sidx[p][0, sl], 1)
        rh[p][0, sl] = lax.shift_right_logical(ridx[p][0, sl], 1)
      pltpu.async_copy(ceproj_hbm.at[pl.ds(pbase + j * (CHUNK // 2), CHUNK // 2)],
                       ce[p], sem_ce[p])
      pltpu.async_copy(xproj_hbm.at[sh[p].at[0]], xs[p], sem_g[p])

    def process(j, p):
      pltpu.make_async_copy(
          ceproj_hbm.at[pl.ds(pbase + j * (CHUNK // 2), CHUNK // 2)],
          ce[p], sem_ce[p]).wait()
      pltpu.make_async_copy(xproj_hbm.at[pl.ds(0, CHUNK)],
                            xs[p], sem_g[p]).wait()

      def group(g, rc):
        sv = sidx[p][0, pl.ds(g * 16, 16)]
        rv = ridx[p][0, pl.ds(g * 16, 16)]
        base_r = g * 16
        base_rp = g * 8
        for lane in range(16):
          r = base_r + lane
          rp = base_rp + lane // 2
          soff = (sv[lane] & 1) * 64
          roff = (rv[lane] & 1) * 64
          zoff = 64 - roff
          for q in range(H // 16):
            cv = ce[p][rp, pl.ds((lane % 2) * 64 + q * 16, 16)]
            xv = xs[p][r, pl.ds(soff + q * 16, 16)]
            he[r, pl.ds(roff + q * 16, 16)] = jnp.maximum(cv + xv, 0.0)
            he[r, pl.ds(zoff + q * 16, 16)] = jnp.zeros((16,), jnp.float32)
        return rc
      lax.fori_loop(0, CHUNK // 16, group, 0)

      pltpu.sync_copy(he, acc.at[rh[p].at[0]], add=True)

    issue(0, 0)

    def two(t, carry):
      j = 2 * t
      issue(j + 1, 1)
      process(j, 0)
      issue(j + 2, 0)
      process(j + 1, 1)
      return carry
    lax.fori_loop(0, (NCHUNK - 1) // 2, two, 0)
    process(NCHUNK - 1, 0)
    plsc.subcore_barrier()

    pltpu.sync_copy(acc.at[pl.ds(s * ROWS_PER_TILE, ROWS_PER_TILE)],
                    out_hbm.at[c, pl.ds(s * ROWS_PER_TILE, ROWS_PER_TILE)])

  return k(xproj2, ceproj2, senders, receivers)


def _xproj_body(x2_ref, w2_ref, g_ref, wg_ref, b_ref, o_ref):
  cst = jnp.dot(g_ref[...], wg_ref[...], preferred_element_type=jnp.float32)
  cst2 = jnp.concatenate([cst + b_ref[...], cst + b_ref[...]], axis=1)
  o_ref[...] = (jnp.dot(x2_ref[...], w2_ref[...],
                        preferred_element_type=jnp.float32) + cst2)


def _ceproj_body(e2_ref, w2_ref, o_ref):
  o_ref[...] = jnp.dot(e2_ref[...], w2_ref[...],
                       preferred_element_type=jnp.float32)


def _node1_body(aggp_ref, x_ref, wa_ref, wx_ref, g_ref, wg_ref, b_ref,
                agg_ref, hv_ref, sums_ref, acc_ref):
  i = pl.program_id(0)
  agg = aggp_ref[0] + aggp_ref[1]
  agg_ref[...] = agg
  const = jnp.dot(g_ref[...], wg_ref[...], preferred_element_type=jnp.float32)
  hv = jnp.maximum(
      jnp.dot(agg, wa_ref[...], preferred_element_type=jnp.float32)
      + jnp.dot(x_ref[...], wx_ref[...], preferred_element_type=jnp.float32)
      + const + b_ref[...], 0.0)
  hv_ref[...] = hv
  part = jnp.concatenate(
      [jnp.sum(agg, axis=0, keepdims=True),
       jnp.sum(hv, axis=0, keepdims=True)], axis=0)

  @pl.when(i == 0)
  def _():
    acc_ref[...] = jnp.zeros_like(acc_ref)

  acc_ref[0:2, 0:H] += part

  @pl.when(i == pl.num_programs(0) - 1)
  def _():
    sums_ref[...] = acc_ref[0:2, 0:H]


def _node2_body(agg_ref, hv_ref, sums_ref, g_ref,
                wg1_ref, wg2_ref, wg3_ref, bgb_ref,
                wa_ref, wv_ref, wgn_ref, b2a_ref, w2b_ref, b2b_ref,
                out_ref, gnew_ref):
  i = pl.program_id(0)
  mean_he = sums_ref[0:1, :] * (1.0 / E)
  mean_hv = sums_ref[1:2, :] * (1.0 / N)
  g_new = jnp.maximum(
      jnp.dot(mean_he, wg1_ref[...], preferred_element_type=jnp.float32)
      + jnp.dot(mean_hv, wg2_ref[...], preferred_element_type=jnp.float32)
      + jnp.dot(g_ref[...], wg3_ref[...], preferred_element_type=jnp.float32)
      + bgb_ref[...], 0.0)
  h2 = jnp.maximum(
      jnp.dot(agg_ref[...], wa_ref[...], preferred_element_type=jnp.float32)
      + jnp.dot(hv_ref[...], wv_ref[...], preferred_element_type=jnp.float32)
      + jnp.dot(g_new, wgn_ref[...], preferred_element_type=jnp.float32)
      + b2a_ref[...], 0.0)
  out_ref[...] = (jnp.dot(h2, w2b_ref[...], preferred_element_type=jnp.float32)
                  + b2b_ref[...])

  @pl.when(i == 0)
  def _():
    gnew_ref[...] = g_new


def _full(shape):
  nd = len(shape)
  return pl.BlockSpec(shape, lambda i: (0,) * nd)


def _blockdiag2(w):
  z = jnp.zeros_like(w)
  return jnp.concatenate([jnp.concatenate([w, z], axis=1),
                          jnp.concatenate([z, w], axis=1)], axis=0)


def kernel(cat_x, cat_e, edge_index, global_attr, W_eb, b_eb, W_nb, b_nb,
           W_gb, b_gb, W_n2a, b_n2a, W_n2b, b_n2b):
  IN_X = cat_x.shape[1]
  IN_E = cat_e.shape[1]
  G = global_attr.shape[0]
  senders = edge_index[0]
  receivers = edge_index[1]
  g_row = global_attr.reshape(1, G)

  W_eb_e = W_eb[:IN_E]
  W_eb_x = W_eb[IN_E:IN_E + IN_X]
  W_eb_g = W_eb[IN_E + IN_X:]

  BN2 = 1000
  xproj2 = pl.pallas_call(
      _xproj_body,
      grid=(N2 // BN2,),
      in_specs=[pl.BlockSpec((BN2, 2 * IN_X), lambda i: (i, 0)),
                _full((2 * IN_X, HP)), _full((1, G)), _full((G, H)),
                _full((1, H))],
      out_specs=pl.BlockSpec((BN2, HP), lambda i: (i, 0)),
      out_shape=jax.ShapeDtypeStruct((N2, HP), jnp.float32),
  )(cat_x.reshape(N2, 2 * IN_X), _blockdiag2(W_eb_x), g_row, W_eb_g,
    b_eb.reshape(1, H))

  BE2 = 4000
  E2 = E // 2
  ceproj2 = pl.pallas_call(
      _ceproj_body,
      grid=(E2 // BE2,),
      in_specs=[pl.BlockSpec((BE2, 2 * IN_E), lambda i: (i, 0)),
                _full((2 * IN_E, HP))],
      out_specs=pl.BlockSpec((BE2, HP), lambda i: (i, 0)),
      out_shape=jax.ShapeDtypeStruct((E2, HP), jnp.float32),
  )(cat_e.reshape(E2, 2 * IN_E), _blockdiag2(W_eb_e))

  agg_packed = _sc_edge_aggregate(xproj2, ceproj2, senders, receivers)
  agg_parts = agg_packed.reshape(NC, 2 * N2_PAD, H)

  W_nb_a = W_nb[:H]
  W_nb_x = W_nb[H:H + IN_X]
  W_nb_g = W_nb[H + IN_X:]
  BN = 2000
  agg, h_v, sums = pl.pallas_call(
      _node1_body,
      grid=(N // BN,),
      in_specs=[pl.BlockSpec((NC, BN, H), lambda i: (0, i, 0)),
                pl.BlockSpec((BN, IN_X), lambda i: (i, 0)),
                _full((H, H)), _full((IN_X, H)), _full((1, G)), _full((G, H)),
                _full((1, H))],
      out_specs=[pl.BlockSpec((BN, H), lambda i: (i, 0)),
                 pl.BlockSpec((BN, H), lambda i: (i, 0)),
                 _full((2, H))],
      out_shape=[jax.ShapeDtypeStruct((N, H), jnp.float32),
                 jax.ShapeDtypeStruct((N, H), jnp.float32),
                 jax.ShapeDtypeStruct((2, H), jnp.float32)],
      scratch_shapes=[pltpu.VMEM((8, 128), jnp.float32)],
  )(agg_parts, cat_x, W_nb_a, W_nb_x, g_row, W_nb_g, b_nb.reshape(1, H))

  OUT = W_n2b.shape[1]
  W_gb1 = W_gb[:H]
  W_gb2 = W_gb[H:2 * H]
  W_gb3 = W_gb[2 * H:]
  W_n2a_a = W_n2a[:H]
  W_n2a_v = W_n2a[H:2 * H]
  W_n2a_g = W_n2a[2 * H:]
  out_nodes, g_new = pl.pallas_call(
      _node2_body,
      grid=(N // BN,),
      in_specs=[pl.BlockSpec((BN, H), lambda i: (i, 0)),
                pl.BlockSpec((BN, H), lambda i: (i, 0)),
                _full((2, H)), _full((1, G)),
                _full((H, G)), _full((H, G)), _full((G, G)), _full((1, G)),
                _full((H, H)), _full((H, H)), _full((G, H)), _full((1, H)),
                _full((H, OUT)), _full((1, OUT))],
      out_specs=[pl.BlockSpec((BN, OUT), lambda i: (i, 0)),
                 _full((1, G))],
      out_shape=[jax.ShapeDtypeStruct((N, OUT), jnp.float32),
                 jax.ShapeDtypeStruct((1, G), jnp.float32)],
  )(agg, h_v, sums, g_row, W_gb1, W_gb2, W_gb3, b_gb.reshape(1, G),
    W_n2a_a, W_n2a_v, W_n2a_g, b_n2a.reshape(1, H),
    W_n2b, b_n2b.reshape(1, OUT))

  return (out_nodes, g_new.reshape(G))
